# baseline (speedup 1.0000x reference)
import jax
import jax.numpy as jnp
from jax.experimental import pallas as pl
from jax.experimental.pallas import tpu as pltpu

_TAU0 = 0.5
_NSLOTS = 8
_SUB = 8


def _body(logits_ref, u_hbm, out_ref, u_buf, sems):
    b = pl.program_id(0)
    nb = pl.num_programs(0)

    @pl.when(b == 0)
    def _prologue():
        for j in range(_NSLOTS - 1):
            pltpu.make_async_copy(u_hbm.at[j], u_buf.at[j], sems.at[j]).start()

    nxt = b + _NSLOTS - 1

    @pl.when(nxt < nb)
    def _prefetch():
        slot = jax.lax.rem(nxt, _NSLOTS)
        pltpu.make_async_copy(u_hbm.at[nxt], u_buf.at[slot], sems.at[slot]).start()

    cur = jax.lax.rem(b, _NSLOTS)
    pltpu.make_async_copy(u_hbm.at[b], u_buf.at[cur], sems.at[cur]).wait()

    l = logits_ref[0]
    u = u_buf[cur]
    e2l = jnp.exp(l * (1.0 / _TAU0))
    t = jnp.log2(u)
    w = e2l[None] / (t * t)
    s = jnp.sum(w, axis=(1, 2), keepdims=True)
    out_ref[0] = jnp.max(w * (1.0 / s), axis=0)


def kernel(logits, uniform):
    B, D = logits.shape
    _, K, _ = uniform.shape
    DL = D // _SUB
    out = pl.pallas_call(
        _body,
        grid=(B,),
        in_specs=[
            pl.BlockSpec((1, _SUB, DL), lambda b: (b, 0, 0)),
            pl.BlockSpec(memory_space=pl.ANY),
        ],
        out_specs=pl.BlockSpec((1, _SUB, DL), lambda b: (b, 0, 0)),
        out_shape=jax.ShapeDtypeStruct((B, _SUB, DL), jnp.float32),
        scratch_shapes=[
            pltpu.VMEM((_NSLOTS, K, _SUB, DL), jnp.float32),
            pltpu.SemaphoreType.DMA((_NSLOTS,)),
        ],
        compiler_params=pltpu.CompilerParams(
            dimension_semantics=("arbitrary",),
            vmem_limit_bytes=100 * 1024 * 1024,
        ),
    )(logits.reshape(B, _SUB, DL), uniform.reshape(B, K, _SUB, DL))
    return out.reshape(B, D)


# speedup vs baseline: 1.2851x; 1.2851x over previous
import jax
import jax.numpy as jnp
from jax.experimental import pallas as pl
from jax.experimental.pallas import tpu as pltpu

_TAU0 = 0.5
_BB = 2


def _body(logits_ref, u_ref, out_ref):
    for i in range(_BB):
        l = logits_ref[i]
        u = u_ref[i]
        e2l = jnp.exp(l * (1.0 / _TAU0))
        t = jnp.log2(u)
        w = e2l / (t * t)
        s = jnp.sum(w, axis=-1, keepdims=True)
        out_ref[i] = jnp.max(w * (1.0 / s), axis=0, keepdims=True)


def kernel(logits, uniform):
    B, D = logits.shape
    _, K, _ = uniform.shape
    out = pl.pallas_call(
        _body,
        grid=(B // _BB,),
        in_specs=[
            pl.BlockSpec((_BB, 1, D), lambda b: (b, 0, 0)),
            pl.BlockSpec((_BB, K, D), lambda b: (b, 0, 0)),
        ],
        out_specs=pl.BlockSpec((_BB, 1, D), lambda b: (b, 0, 0)),
        out_shape=jax.ShapeDtypeStruct((B, 1, D), jnp.float32),
        compiler_params=pltpu.CompilerParams(
            dimension_semantics=("arbitrary",),
            vmem_limit_bytes=100 * 1024 * 1024,
        ),
    )(logits.reshape(B, 1, D), uniform)
    return out.reshape(B, D)


# speedup vs baseline: 1.3065x; 1.0166x over previous
import jax
import jax.numpy as jnp
from jax.experimental import pallas as pl
from jax.experimental.pallas import tpu as pltpu

_TAU0 = 0.5
_BB = 4


def _body(logits_ref, u_ref, out_ref):
    for i in range(_BB):
        l = logits_ref[i]
        u = u_ref[i]
        e2l = jnp.exp(l * (1.0 / _TAU0))
        t = jnp.log(u)
        w = e2l / (t * t)
        s = jnp.sum(w, axis=-1, keepdims=True)
        out_ref[i] = jnp.max(w * (1.0 / s), axis=0, keepdims=True)


def kernel(logits, uniform):
    B, D = logits.shape
    _, K, _ = uniform.shape
    out = pl.pallas_call(
        _body,
        grid=(B // _BB,),
        in_specs=[
            pl.BlockSpec((_BB, 1, D), lambda b: (b, 0, 0)),
            pl.BlockSpec((_BB, K, D), lambda b: (b, 0, 0)),
        ],
        out_specs=pl.BlockSpec((_BB, 1, D), lambda b: (b, 0, 0)),
        out_shape=jax.ShapeDtypeStruct((B, 1, D), jnp.float32),
        compiler_params=pltpu.CompilerParams(
            dimension_semantics=("arbitrary",),
            vmem_limit_bytes=100 * 1024 * 1024,
        ),
    )(logits.reshape(B, 1, D), uniform)
    return out.reshape(B, D)


# speedup vs baseline: 1.3123x; 1.0045x over previous
import jax
import jax.numpy as jnp
from jax.experimental import pallas as pl
from jax.experimental.pallas import tpu as pltpu

_TAU0 = 0.5
_BB = 2


def _body(logits_ref, u_ref, out_ref):
    for i in range(_BB):
        l = logits_ref[i]
        u = u_ref[i]
        e2l = jnp.exp(l * (1.0 / _TAU0))
        t = jnp.log(u)
        w = e2l / (t * t)
        s = jnp.sum(w, axis=-1, keepdims=True)
        out_ref[i] = jnp.max(w * (1.0 / s), axis=0, keepdims=True)


def kernel(logits, uniform):
    B, D = logits.shape
    _, K, _ = uniform.shape
    out = pl.pallas_call(
        _body,
        grid=(B // _BB,),
        in_specs=[
            pl.BlockSpec((_BB, 1, D), lambda b: (b, 0, 0)),
            pl.BlockSpec((_BB, K, D), lambda b: (b, 0, 0)),
        ],
        out_specs=pl.BlockSpec((_BB, 1, D), lambda b: (b, 0, 0)),
        out_shape=jax.ShapeDtypeStruct((B, 1, D), jnp.float32),
        compiler_params=pltpu.CompilerParams(
            dimension_semantics=("arbitrary",),
            vmem_limit_bytes=100 * 1024 * 1024,
        ),
    )(logits.reshape(B, 1, D), uniform)
    return out.reshape(B, D)


# speedup vs baseline: 1.3325x; 1.0154x over previous
import jax
import jax.numpy as jnp
from jax.experimental import pallas as pl
from jax.experimental.pallas import tpu as pltpu

_TAU0 = 0.5
_BB = 2


def _body(logits_ref, u_ref, out_ref):
    for i in range(_BB):
        l = logits_ref[i]
        u = u_ref[i]
        e2l = jnp.exp(l * (1.0 / _TAU0))
        t = jnp.log(u)
        w = e2l / (t * t)
        s = jnp.sum(w, axis=-1, keepdims=True)
        out_ref[i] = jnp.max(w * (1.0 / s), axis=0, keepdims=True)


def kernel(logits, uniform):
    B, D = logits.shape
    _, K, _ = uniform.shape
    out = pl.pallas_call(
        _body,
        grid=(B // _BB,),
        in_specs=[
            pl.BlockSpec((_BB, 1, D), lambda b: (b, 0, 0)),
            pl.BlockSpec((_BB, K, D), lambda b: (b, 0, 0)),
        ],
        out_specs=pl.BlockSpec((_BB, 1, D), lambda b: (b, 0, 0)),
        out_shape=jax.ShapeDtypeStruct((B, 1, D), jnp.float32),
        compiler_params=pltpu.CompilerParams(
            dimension_semantics=("parallel",),
            vmem_limit_bytes=100 * 1024 * 1024,
        ),
    )(logits.reshape(B, 1, D), uniform)
    return out.reshape(B, D)
